# Initial kernel scaffold; baseline (speedup 1.0000x reference)
import jax
import jax.numpy as jnp
from jax.experimental import pallas as pl
from jax.experimental.pallas import tpu as pltpu

_N = 100000
_D = 512
_G = 128
_EPS = 1e-07
_B = 1000
_NB = _N // _B


def _body(batch_ref, x_ref, o_ref, sums_ref, counts_ref, scale_ref):
    k = pl.program_id(0)
    j = pl.program_id(1)
    b = batch_ref[0, 0, :]
    onehot = (b[:, None] == jax.lax.broadcasted_iota(jnp.int32, (_B, _G), 1)
              ).astype(jnp.float32)

    @pl.when(jnp.logical_and(k == 0, j == 0))
    def _init():
        sums_ref[...] = jnp.zeros_like(sums_ref)
        counts_ref[...] = jnp.zeros_like(counts_ref)

    @pl.when(k == 0)
    def _acc():
        x = x_ref[...]
        sq = x * x
        sums_ref[...] += jax.lax.dot_general(
            onehot, sq, (((0,), (0,)), ((), ())),
            preferred_element_type=jnp.float32)
        counts_ref[...] += jnp.sum(onehot, axis=0)[:, None]

    @pl.when(jnp.logical_and(k == 1, j == 0))
    def _scale():
        cnt = jnp.maximum(counts_ref[...], 1.0)
        scale_ref[...] = jax.lax.rsqrt(sums_ref[...] / cnt + _EPS)

    @pl.when(k == 1)
    def _norm():
        gathered = jnp.dot(onehot, scale_ref[...],
                           preferred_element_type=jnp.float32)
        o_ref[...] = x_ref[...] * gathered


@jax.jit
def kernel(input, batch, num_graphs):
    del num_graphs
    batch3 = batch.reshape(_NB, 1, _B).astype(jnp.int32)
    out = pl.pallas_call(
        _body,
        grid=(2, _NB),
        in_specs=[
            pl.BlockSpec((1, 1, _B), lambda k, j: (j, 0, 0)),
            pl.BlockSpec((_B, _D), lambda k, j: (j, 0)),
        ],
        out_specs=pl.BlockSpec((_B, _D), lambda k, j: (j, 0)),
        out_shape=jax.ShapeDtypeStruct((_N, _D), jnp.float32),
        scratch_shapes=[
            pltpu.VMEM((_G, _D), jnp.float32),
            pltpu.VMEM((_G, 1), jnp.float32),
            pltpu.VMEM((_G, _D), jnp.float32),
        ],
        compiler_params=pltpu.CompilerParams(
            dimension_semantics=("arbitrary", "arbitrary")),
    )(batch3, input)
    return out


# speedup vs baseline: 5.7347x; 5.7347x over previous
import jax
import jax.numpy as jnp
from jax.experimental import pallas as pl
from jax.experimental.pallas import tpu as pltpu

_N = 100000
_D = 512
_G = 128
_EPS = 1e-07
_B = 1000
_NB = _N // _B


def _onehot(batch_ref):
    b = batch_ref[0, 0, :]
    return (b[:, None] == jax.lax.broadcasted_iota(jnp.int32, (_B, _G), 1)
            ).astype(jnp.float32)


def _reduce_body(batch_ref, x_ref, sums_ref, counts_ref):
    j = pl.program_id(0)
    onehot = _onehot(batch_ref)

    @pl.when(j == 0)
    def _init():
        sums_ref[...] = jnp.zeros_like(sums_ref)
        counts_ref[...] = jnp.zeros_like(counts_ref)

    x = x_ref[...]
    sums_ref[...] += jax.lax.dot_general(
        onehot, x * x, (((0,), (0,)), ((), ())),
        preferred_element_type=jnp.float32)
    counts_ref[...] += jnp.sum(onehot, axis=0)[:, None]


def _norm_body(batch_ref, x_ref, sums_ref, counts_ref, o_ref, scale_ref):
    j = pl.program_id(0)

    @pl.when(j == 0)
    def _scale():
        cnt = jnp.maximum(counts_ref[...], 1.0)
        scale_ref[...] = jax.lax.rsqrt(sums_ref[...] / cnt + _EPS)

    gathered = jnp.dot(_onehot(batch_ref), scale_ref[...],
                       preferred_element_type=jnp.float32)
    o_ref[...] = x_ref[...] * gathered


@jax.jit
def kernel(input, batch, num_graphs):
    del num_graphs
    batch3 = batch.reshape(_NB, 1, _B).astype(jnp.int32)
    sums, counts = pl.pallas_call(
        _reduce_body,
        grid=(_NB,),
        in_specs=[
            pl.BlockSpec((1, 1, _B), lambda j: (j, 0, 0)),
            pl.BlockSpec((_B, _D), lambda j: (j, 0)),
        ],
        out_specs=[
            pl.BlockSpec((_G, _D), lambda j: (0, 0)),
            pl.BlockSpec((_G, 1), lambda j: (0, 0)),
        ],
        out_shape=[
            jax.ShapeDtypeStruct((_G, _D), jnp.float32),
            jax.ShapeDtypeStruct((_G, 1), jnp.float32),
        ],
        compiler_params=pltpu.CompilerParams(
            dimension_semantics=("arbitrary",)),
    )(batch3, input)
    out = pl.pallas_call(
        _norm_body,
        grid=(_NB,),
        in_specs=[
            pl.BlockSpec((1, 1, _B), lambda j: (j, 0, 0)),
            pl.BlockSpec((_B, _D), lambda j: (j, 0)),
            pl.BlockSpec((_G, _D), lambda j: (0, 0)),
            pl.BlockSpec((_G, 1), lambda j: (0, 0)),
        ],
        out_specs=pl.BlockSpec((_B, _D), lambda j: (j, 0)),
        out_shape=jax.ShapeDtypeStruct((_N, _D), jnp.float32),
        scratch_shapes=[pltpu.VMEM((_G, _D), jnp.float32)],
        compiler_params=pltpu.CompilerParams(
            dimension_semantics=("arbitrary",)),
    )(batch3, input, sums, counts)
    return out


# speedup vs baseline: 7.1121x; 1.2402x over previous
import jax
import jax.numpy as jnp
from jax.experimental import pallas as pl
from jax.experimental.pallas import tpu as pltpu

_N = 100000
_D = 512
_G = 128
_EPS = 1e-07
_B = 2000
_NB = _N // _B


def _onehot(batch_ref):
    b = batch_ref[0, 0, :]
    return (b[:, None] == jax.lax.broadcasted_iota(jnp.int32, (_B, _G), 1)
            ).astype(jnp.float32)


def _reduce_body(batch_ref, x_ref, sums_ref, counts_ref):
    j = pl.program_id(0)
    onehot = _onehot(batch_ref)

    @pl.when(j == 0)
    def _init():
        sums_ref[...] = jnp.zeros_like(sums_ref)
        counts_ref[...] = jnp.zeros_like(counts_ref)

    x = x_ref[...]
    sums_ref[...] += jax.lax.dot_general(
        onehot, x * x, (((0,), (0,)), ((), ())),
        preferred_element_type=jnp.float32)
    counts_ref[...] += jnp.sum(onehot, axis=0)[:, None]


def _norm_body(batch_ref, x_ref, sums_ref, counts_ref, o_ref, scale_ref):
    j = pl.program_id(0)

    @pl.when(j == 0)
    def _scale():
        cnt = jnp.maximum(counts_ref[...], 1.0)
        scale_ref[...] = jax.lax.rsqrt(sums_ref[...] / cnt + _EPS)

    gathered = jnp.dot(_onehot(batch_ref), scale_ref[...],
                       preferred_element_type=jnp.float32)
    o_ref[...] = x_ref[...] * gathered


@jax.jit
def kernel(input, batch, num_graphs):
    del num_graphs
    batch3 = batch.reshape(_NB, 1, _B).astype(jnp.int32)
    sums, counts = pl.pallas_call(
        _reduce_body,
        grid=(_NB,),
        in_specs=[
            pl.BlockSpec((1, 1, _B), lambda j: (j, 0, 0)),
            pl.BlockSpec((_B, _D), lambda j: (j, 0)),
        ],
        out_specs=[
            pl.BlockSpec((_G, _D), lambda j: (0, 0)),
            pl.BlockSpec((_G, 1), lambda j: (0, 0)),
        ],
        out_shape=[
            jax.ShapeDtypeStruct((_G, _D), jnp.float32),
            jax.ShapeDtypeStruct((_G, 1), jnp.float32),
        ],
        compiler_params=pltpu.CompilerParams(
            dimension_semantics=("arbitrary",)),
    )(batch3, input)
    out = pl.pallas_call(
        _norm_body,
        grid=(_NB,),
        in_specs=[
            pl.BlockSpec((1, 1, _B), lambda j: (j, 0, 0)),
            pl.BlockSpec((_B, _D), lambda j: (j, 0)),
            pl.BlockSpec((_G, _D), lambda j: (0, 0)),
            pl.BlockSpec((_G, 1), lambda j: (0, 0)),
        ],
        out_specs=pl.BlockSpec((_B, _D), lambda j: (j, 0)),
        out_shape=jax.ShapeDtypeStruct((_N, _D), jnp.float32),
        scratch_shapes=[pltpu.VMEM((_G, _D), jnp.float32)],
        compiler_params=pltpu.CompilerParams(
            dimension_semantics=("arbitrary",)),
    )(batch3, input, sums, counts)
    return out


# speedup vs baseline: 7.7467x; 1.0892x over previous
import jax
import jax.numpy as jnp
from jax.experimental import pallas as pl
from jax.experimental.pallas import tpu as pltpu

_N = 100000
_D = 512
_G = 128
_EPS = 1e-07
_B = 4000
_NB = _N // _B


def _onehot(batch_ref):
    b = batch_ref[0, 0, :]
    return (b[:, None] == jax.lax.broadcasted_iota(jnp.int32, (_B, _G), 1)
            ).astype(jnp.float32)


def _reduce_body(batch_ref, x_ref, sums_ref, counts_ref):
    j = pl.program_id(0)
    onehot = _onehot(batch_ref)

    @pl.when(j == 0)
    def _init():
        sums_ref[...] = jnp.zeros_like(sums_ref)
        counts_ref[...] = jnp.zeros_like(counts_ref)

    x = x_ref[...]
    sums_ref[...] += jax.lax.dot_general(
        onehot, x * x, (((0,), (0,)), ((), ())),
        preferred_element_type=jnp.float32)
    counts_ref[...] += jnp.sum(onehot, axis=0)[:, None]


def _norm_body(batch_ref, x_ref, sums_ref, counts_ref, o_ref, scale_ref):
    j = pl.program_id(0)

    @pl.when(j == 0)
    def _scale():
        cnt = jnp.maximum(counts_ref[...], 1.0)
        scale_ref[...] = jax.lax.rsqrt(sums_ref[...] / cnt + _EPS)

    gathered = jnp.dot(_onehot(batch_ref), scale_ref[...],
                       preferred_element_type=jnp.float32)
    o_ref[...] = x_ref[...] * gathered


@jax.jit
def kernel(input, batch, num_graphs):
    del num_graphs
    batch3 = batch.reshape(_NB, 1, _B).astype(jnp.int32)
    sums, counts = pl.pallas_call(
        _reduce_body,
        grid=(_NB,),
        in_specs=[
            pl.BlockSpec((1, 1, _B), lambda j: (j, 0, 0)),
            pl.BlockSpec((_B, _D), lambda j: (j, 0)),
        ],
        out_specs=[
            pl.BlockSpec((_G, _D), lambda j: (0, 0)),
            pl.BlockSpec((_G, 1), lambda j: (0, 0)),
        ],
        out_shape=[
            jax.ShapeDtypeStruct((_G, _D), jnp.float32),
            jax.ShapeDtypeStruct((_G, 1), jnp.float32),
        ],
        compiler_params=pltpu.CompilerParams(
            dimension_semantics=("arbitrary",)),
    )(batch3, input)
    out = pl.pallas_call(
        _norm_body,
        grid=(_NB,),
        in_specs=[
            pl.BlockSpec((1, 1, _B), lambda j: (j, 0, 0)),
            pl.BlockSpec((_B, _D), lambda j: (j, 0)),
            pl.BlockSpec((_G, _D), lambda j: (0, 0)),
            pl.BlockSpec((_G, 1), lambda j: (0, 0)),
        ],
        out_specs=pl.BlockSpec((_B, _D), lambda j: (j, 0)),
        out_shape=jax.ShapeDtypeStruct((_N, _D), jnp.float32),
        scratch_shapes=[pltpu.VMEM((_G, _D), jnp.float32)],
        compiler_params=pltpu.CompilerParams(
            dimension_semantics=("arbitrary",)),
    )(batch3, input, sums, counts)
    return out


# speedup vs baseline: 7.8226x; 1.0098x over previous
import jax
import jax.numpy as jnp
from jax.experimental import pallas as pl
from jax.experimental.pallas import tpu as pltpu

_N = 100000
_D = 512
_G = 128
_EPS = 1e-07
_B = 5000
_NB = _N // _B


def _onehot(batch_ref):
    b = batch_ref[0, 0, :]
    return (b[:, None] == jax.lax.broadcasted_iota(jnp.int32, (_B, _G), 1)
            ).astype(jnp.float32)


def _reduce_body(batch_ref, x_ref, sums_ref, counts_ref):
    j = pl.program_id(0)
    onehot = _onehot(batch_ref)

    @pl.when(j == 0)
    def _init():
        sums_ref[...] = jnp.zeros_like(sums_ref)
        counts_ref[...] = jnp.zeros_like(counts_ref)

    x = x_ref[...]
    sums_ref[...] += jax.lax.dot_general(
        onehot, x * x, (((0,), (0,)), ((), ())),
        preferred_element_type=jnp.float32)
    counts_ref[...] += jnp.sum(onehot, axis=0)[:, None]


def _norm_body(batch_ref, x_ref, sums_ref, counts_ref, o_ref, scale_ref):
    j = pl.program_id(0)

    @pl.when(j == 0)
    def _scale():
        cnt = jnp.maximum(counts_ref[...], 1.0)
        scale_ref[...] = jax.lax.rsqrt(sums_ref[...] / cnt + _EPS)

    gathered = jnp.dot(_onehot(batch_ref), scale_ref[...],
                       preferred_element_type=jnp.float32)
    o_ref[...] = x_ref[...] * gathered


@jax.jit
def kernel(input, batch, num_graphs):
    del num_graphs
    batch3 = batch.reshape(_NB, 1, _B).astype(jnp.int32)
    sums, counts = pl.pallas_call(
        _reduce_body,
        grid=(_NB,),
        in_specs=[
            pl.BlockSpec((1, 1, _B), lambda j: (j, 0, 0)),
            pl.BlockSpec((_B, _D), lambda j: (j, 0)),
        ],
        out_specs=[
            pl.BlockSpec((_G, _D), lambda j: (0, 0)),
            pl.BlockSpec((_G, 1), lambda j: (0, 0)),
        ],
        out_shape=[
            jax.ShapeDtypeStruct((_G, _D), jnp.float32),
            jax.ShapeDtypeStruct((_G, 1), jnp.float32),
        ],
        compiler_params=pltpu.CompilerParams(
            dimension_semantics=("arbitrary",)),
    )(batch3, input)
    out = pl.pallas_call(
        _norm_body,
        grid=(_NB,),
        in_specs=[
            pl.BlockSpec((1, 1, _B), lambda j: (j, 0, 0)),
            pl.BlockSpec((_B, _D), lambda j: (j, 0)),
            pl.BlockSpec((_G, _D), lambda j: (0, 0)),
            pl.BlockSpec((_G, 1), lambda j: (0, 0)),
        ],
        out_specs=pl.BlockSpec((_B, _D), lambda j: (j, 0)),
        out_shape=jax.ShapeDtypeStruct((_N, _D), jnp.float32),
        scratch_shapes=[pltpu.VMEM((_G, _D), jnp.float32)],
        compiler_params=pltpu.CompilerParams(
            dimension_semantics=("arbitrary",)),
    )(batch3, input, sums, counts)
    return out
